# Initial kernel scaffold; baseline (speedup 1.0000x reference)
import jax
import jax.numpy as jnp
from jax.experimental import pallas as pl


def kernel(embeddings, cu_seqlens, max_seqlen, W1, b1, W2, b2):
    raise NotImplementedError("write your pallas kernel here")


# speedup vs baseline: 5.6459x; 5.6459x over previous
import functools

import jax
import jax.numpy as jnp
from jax import lax
from jax.experimental import pallas as pl
from jax.experimental.pallas import tpu as pltpu
from jax.experimental.pallas import tpu_sc as plsc

N_TOK = 32768
D = 768
NSEG = 16
H = 256

NC = 2
NS = 16
NW = NC * NS
ROWS_W = N_TOK // NW
CHUNK = 64
NCHUNK = ROWS_W // CHUNK
LANES = 16
NV = D // LANES


def _sc_body(emb, bounds, partials, bounds_v, buf2, acc_v, sem0, sem1):
  cid = lax.axis_index("c")
  sid = lax.axis_index("s")
  wid = sid * NC + cid
  base = wid * ROWS_W

  pltpu.sync_copy(bounds.at[wid], bounds_v)
  lov = bounds_v[pl.ds(0, 16)]
  hiv = bounds_v[pl.ds(16, 16)]
  jlov = bounds_v[pl.ds(32, 16)]
  jhiv = bounds_v[pl.ds(48, 16)]
  los = [lov[k] for k in range(NSEG)]
  his = [hiv[k] for k in range(NSEG)]

  def select_at(vals, j):
    out = vals[0]
    for k in range(1, NSEG):
      out = jnp.where(j == k, vals[k], out)
    return out

  def zero_body(i, _):
    acc_v[pl.ds(i * LANES, LANES)] = jnp.zeros((LANES,), jnp.float32)
    return 0
  lax.fori_loop(0, NSEG * D // LANES, zero_body, 0)

  sems = (sem0, sem1)

  def start(c, b):
    pltpu.async_copy(emb.at[pl.ds(base + c * CHUNK, CHUNK)],
                     buf2.at[pl.ds(b * CHUNK, CHUNK)], sems[b])

  def wait(b):
    pltpu.make_async_copy(emb.at[pl.ds(base, CHUNK)],
                          buf2.at[pl.ds(b * CHUNK, CHUNK)], sems[b]).wait()

  start(0, 0)
  start(1, 1)

  def process(c0, rbase, jlo, jhi):
    def seg_body(j, _):
      lo = select_at(los, j)
      hi = select_at(his, j)
      r0 = jnp.maximum(lo - c0, 0)
      r1 = jnp.minimum(hi - c0, CHUNK)
      r1 = jnp.maximum(r1, r0)

      accs0 = tuple(acc_v[pl.ds(j * D + k * LANES, LANES)]
                    for k in range(NV))

      @functools.partial(plsc.parallel_loop(r0, r1, carry=accs0))
      def row_final(r, accs):
        return tuple(
            accs[k] + buf2[rbase + r, pl.ds(k * LANES, LANES)]
            for k in range(NV))

      for k in range(NV):
        acc_v[pl.ds(j * D + k * LANES, LANES)] = row_final[k]
      return 0

    lax.fori_loop(jlo, jhi, seg_body, 0)

  for c in range(NCHUNK):
    b = c % 2
    wait(b)
    process(c * CHUNK, b * CHUNK, jlov[c], jhiv[c])
    if c + 2 < NCHUNK:
      start(c + 2, b)

  pltpu.sync_copy(acc_v, partials.at[wid])


@functools.partial(jax.jit, static_argnames=())
def _sc_segment_sums(embeddings, bounds):
  mesh = plsc.VectorSubcoreMesh(
      core_axis_name="c", subcore_axis_name="s", num_cores=NC, num_subcores=NS)
  return pl.kernel(
      _sc_body,
      out_type=jax.ShapeDtypeStruct((NW, NSEG * D), jnp.float32),
      mesh=mesh,
      scratch_types=[
          pltpu.VMEM((64,), jnp.int32),
          pltpu.VMEM((2 * CHUNK, D), jnp.float32),
          pltpu.VMEM((NSEG * D,), jnp.float32),
          pltpu.SemaphoreType.DMA,
          pltpu.SemaphoreType.DMA,
      ],
  )(embeddings, bounds)


def _tc_body(partials_ref, cu_ref, w1_ref, b1_ref, w2_ref, b2_ref, out_ref):
  total = jnp.zeros((NSEG, D), jnp.float32)
  for i in range(NW):
    total = total + partials_ref[pl.ds(i * NSEG, NSEG), :]

  ridx = lax.broadcasted_iota(jnp.int32, (NSEG, 1), 0)
  inv = jnp.zeros((NSEG, 1), jnp.float32)
  for j in range(NSEG):
    cnt = (cu_ref[j + 1] - cu_ref[j]).astype(jnp.float32)
    inv = jnp.where(ridx == j, 1.0 / cnt, inv)

  pooled = total * inv
  h = jnp.maximum(
      jnp.dot(pooled, w1_ref[...], preferred_element_type=jnp.float32)
      + b1_ref[...], 0.0)
  out = jnp.sum(h * w2_ref[...], axis=1, keepdims=True) + b2_ref[...]
  out_ref[...] = out


def _tc_head(partials2d, cu_seqlens, W1, b1, W2, b2):
  return pl.pallas_call(
      _tc_body,
      out_shape=jax.ShapeDtypeStruct((NSEG, 1), jnp.float32),
      in_specs=[
          pl.BlockSpec(memory_space=pltpu.VMEM),
          pl.BlockSpec(memory_space=pltpu.SMEM),
          pl.BlockSpec(memory_space=pltpu.VMEM),
          pl.BlockSpec(memory_space=pltpu.VMEM),
          pl.BlockSpec(memory_space=pltpu.VMEM),
          pl.BlockSpec(memory_space=pltpu.VMEM),
      ],
      out_specs=pl.BlockSpec(memory_space=pltpu.VMEM),
  )(partials2d, cu_seqlens, W1, b1.reshape(1, H), W2.reshape(1, H),
    b2.reshape(1, 1))


def kernel(embeddings, cu_seqlens, max_seqlen, W1, b1, W2, b2):
  starts = (jnp.arange(NW, dtype=jnp.int32) * ROWS_W)[:, None]
  clipped = jnp.clip(cu_seqlens[None, :].astype(jnp.int32) - starts, 0, ROWS_W)
  lo = clipped[:, :NSEG]
  hi = clipped[:, 1:NSEG + 1]

  c0 = (jnp.arange(NCHUNK, dtype=jnp.int32) * CHUNK)[None, :, None]
  inter = (jnp.maximum(lo[:, None, :], c0)
           < jnp.minimum(hi[:, None, :], c0 + CHUNK))
  jlo = jnp.argmax(inter, axis=2).astype(jnp.int32)
  jhi = NSEG - jnp.argmax(inter[:, :, ::-1], axis=2).astype(jnp.int32)

  bounds = jnp.concatenate([lo, hi, jlo, jhi], axis=1)

  partials = _sc_segment_sums(embeddings, bounds)
  partials2d = partials.reshape(NW * NSEG, D)

  out = _tc_head(partials2d, cu_seqlens, W1, b1, W2, b2)
  return out.squeeze(-1)
